# baseline (speedup 1.0000x reference)
import functools

import jax
import jax.numpy as jnp
from jax import lax
from jax.experimental import pallas as pl
from jax.experimental.pallas import tpu as pltpu
from jax.experimental.pallas import tpu_sc as plsc

D_MODEL = 64
N_OBS = 200
N_FEAT = 128
NJ = D_MODEL // 16

NUM_CORES = 2
NUM_SUBCORES = 16
NW = NUM_CORES * NUM_SUBCORES

NLC = 8
NBC = 4
BLK = 512


@functools.lru_cache(maxsize=None)
def _build(B: int, L: int):
    lc_size = L // NLC
    bc_size = B // NBC
    sub = bc_size // BLK
    steps = lc_size * sub
    assert L % NLC == 0 and B % NBC == 0 and bc_size % BLK == 0
    assert steps % 2 == 0 and BLK % 16 == 0

    mesh = plsc.VectorSubcoreMesh(
        core_axis_name="c", subcore_axis_name="s",
        num_cores=NUM_CORES, num_subcores=NUM_SUBCORES)

    @functools.partial(
        pl.kernel,
        out_type=jax.ShapeDtypeStruct((L, D_MODEL, B), jnp.float32),
        mesh=mesh,
        compiler_params=pltpu.CompilerParams(needs_layout_passes=False),
        scratch_types=[
            pltpu.VMEM((N_OBS * D_MODEL,), jnp.float32),
            pltpu.VMEM((N_FEAT * D_MODEL,), jnp.float32),
            pltpu.VMEM((D_MODEL,), jnp.float32),
            pltpu.VMEM((D_MODEL,), jnp.float32),
            pltpu.VMEM((2 * BLK,), jnp.int32),
            pltpu.VMEM((2 * BLK,), jnp.int32),
            pltpu.VMEM((2 * BLK,), jnp.float32),
            pltpu.VMEM((BLK * D_MODEL,), jnp.float32),
            pltpu.VMEM((2, D_MODEL, BLK), jnp.float32),
            pltpu.SemaphoreType.DMA,
            pltpu.SemaphoreType.DMA,
            pltpu.SemaphoreType.DMA,
            pltpu.SemaphoreType.DMA,
        ],
    )
    def embed(val_h, obs_h, feat_h, tab_obs_h, tab_feat_h, w_h, bias_h,
              out_h, tab_o, tab_f, w_v, b_v, obs_v, feat_v, val_v,
              skew_v, out_v, sem_in0, sem_in1, sem_out0, sem_out1):
        wid = lax.axis_index("s") * NUM_CORES + lax.axis_index("c")
        lc = wid // NBC
        bq = wid % NBC
        l0 = lc * lc_size
        b0 = bq * bc_size
        sems_in = (sem_in0, sem_in1)
        sems_out = (sem_out0, sem_out1)

        pltpu.sync_copy(tab_obs_h, tab_o)
        pltpu.sync_copy(tab_feat_h, tab_f)
        pltpu.sync_copy(w_h, w_v)
        pltpu.sync_copy(bias_h, b_v)

        w_regs = [w_v[pl.ds(16 * j, 16)] for j in range(NJ)]
        b_regs = [b_v[pl.ds(16 * j, 16)] for j in range(NJ)]

        def bias_body(r, c):
            for j in range(NJ):
                off = r * D_MODEL + 16 * j
                tab_o[pl.ds(off, 16)] = tab_o[pl.ds(off, 16)] + b_regs[j]
            return c

        lax.fori_loop(0, N_OBS, bias_body, 0)

        iota16 = lax.iota(jnp.int32, 16)

        def tok_off(s):
            return (l0 + s // sub) * B + b0 + (s % sub) * BLK

        def in_copies(s, b):
            row0 = tok_off(s)
            return (
                pltpu.make_async_copy(obs_h.at[pl.ds(row0, BLK)],
                                      obs_v.at[pl.ds(b * BLK, BLK)],
                                      sems_in[b]),
                pltpu.make_async_copy(feat_h.at[pl.ds(row0, BLK)],
                                      feat_v.at[pl.ds(b * BLK, BLK)],
                                      sems_in[b]),
                pltpu.make_async_copy(val_h.at[pl.ds(row0, BLK)],
                                      val_v.at[pl.ds(b * BLK, BLK)],
                                      sems_in[b]),
            )

        def out_copy(s, b):
            lg = l0 + s // sub
            bcol = b0 + (s % sub) * BLK
            return pltpu.make_async_copy(
                out_v.at[b],
                out_h.at[lg, :, pl.ds(bcol, BLK)],
                sems_out[b])

        for c in in_copies(0, 0):
            c.start()

        def pair_body(g, carry):
            for b in range(2):
                s = g * 2 + b

                @pl.when(s + 1 < steps)
                def _():
                    for c in in_copies(s + 1, 1 - b):
                        c.start()

                for c in in_copies(s, b):
                    c.wait()

                @pl.when(s >= 2)
                def _():
                    out_copy(s - 2, b).wait()

                def grp_body(gi, carry2):
                    t0b = b * BLK + gi * 16
                    o16 = obs_v[pl.ds(t0b, 16)] * D_MODEL
                    f16 = feat_v[pl.ds(t0b, 16)] * D_MODEL
                    v16 = val_v[pl.ds(t0b, 16)]
                    t0 = gi * 16

                    def load_tok(ob, fb):
                        return (
                            [tab_o[pl.ds(ob + 16 * j, 16)] for j in range(NJ)],
                            [tab_f[pl.ds(fb + 16 * j, 16)] for j in range(NJ)],
                        )

                    PF = 2
                    obase = [None] * 16
                    fbase = [None] * 16
                    for k in range(PF):
                        obase[k] = o16[k]
                        fbase[k] = f16[k]
                    ro, rf = load_tok(obase[0], fbase[0])
                    for k in range(16):
                        if k + PF < 16:
                            obase[k + PF] = o16[k + PF]
                            fbase[k + PF] = f16[k + PF]
                        if k + 1 < 16:
                            ro_n, rf_n = load_tok(obase[k + 1], fbase[k + 1])
                        vb = v16[k]
                        t = t0 + k
                        tb = t * D_MODEL
                        for j in range(NJ):
                            rot = (iota16 + (k + 16 * j)) & (D_MODEL - 1)
                            plsc.store_scatter(
                                skew_v, [rot + tb],
                                ro[j] + rf[j] + vb * w_regs[j])
                        if k + 1 < 16:
                            ro, rf = ro_n, rf_n
                    return carry2

                lax.fori_loop(0, BLK // 16, grp_body, 0)

                NG = 4

                def unskew_body(tg, carry2):
                    tvs = [(tg * 16 * NG + 16 * g + iota16) * D_MODEL
                           for g in range(NG)]
                    rot0 = iota16 & (D_MODEL - 1)
                    cur = [plsc.load_gather(skew_v, [tvs[g] + rot0])
                           for g in range(NG)]
                    for d in range(D_MODEL):
                        if d + 1 < D_MODEL:
                            rot_n = (iota16 + (d + 1)) & (D_MODEL - 1)
                            nxt = [plsc.load_gather(skew_v, [tvs[g] + rot_n])
                                   for g in range(NG)]
                        for g in range(NG):
                            out_v[b, d, pl.ds(tg * 16 * NG + 16 * g, 16)] = (
                                cur[g])
                        if d + 1 < D_MODEL:
                            cur = nxt
                    return carry2

                lax.fori_loop(0, BLK // (16 * NG), unskew_body, 0)

                out_copy(s, b).start()
            return carry

        lax.fori_loop(0, steps // 2, pair_body, 0)
        out_copy(steps - 2, 0).wait()
        out_copy(steps - 1, 1).wait()

    return embed


def kernel(val, obs_idx, feat_idx, W_val, b_val, emb_obs, emb_feat):
    B, L, _ = val.shape
    T = B * L
    val_f = val.reshape(B, L).T.reshape(T).astype(jnp.float32)
    obs_f = obs_idx.T.reshape(T).astype(jnp.int32)
    feat_f = feat_idx.T.reshape(T).astype(jnp.int32)
    w_f = W_val.reshape(D_MODEL).astype(jnp.float32)
    b_f = b_val.reshape(D_MODEL).astype(jnp.float32)
    out_t = _build(B, L)(val_f, obs_f, feat_f,
                         emb_obs.astype(jnp.float32).reshape(N_OBS * D_MODEL),
                         emb_feat.astype(jnp.float32).reshape(N_FEAT * D_MODEL),
                         w_f, b_f)
    return jnp.transpose(out_t, (2, 0, 1))


# speedup vs baseline: 1.0147x; 1.0147x over previous
import functools

import jax
import jax.numpy as jnp
from jax import lax
from jax.experimental import pallas as pl
from jax.experimental.pallas import tpu as pltpu
from jax.experimental.pallas import tpu_sc as plsc

D_MODEL = 64
N_OBS = 200
N_FEAT = 128
NJ = D_MODEL // 16

NUM_CORES = 2
NUM_SUBCORES = 16
NW = NUM_CORES * NUM_SUBCORES

NLC = 8
NBC = 4
BLK = 512


@functools.lru_cache(maxsize=None)
def _build(B: int, L: int):
    lc_size = L // NLC
    bc_size = B // NBC
    sub = bc_size // BLK
    steps = lc_size * sub
    assert L % NLC == 0 and B % NBC == 0 and bc_size % BLK == 0
    assert steps % 2 == 0 and BLK % 16 == 0

    mesh = plsc.VectorSubcoreMesh(
        core_axis_name="c", subcore_axis_name="s",
        num_cores=NUM_CORES, num_subcores=NUM_SUBCORES)

    @functools.partial(
        pl.kernel,
        out_type=jax.ShapeDtypeStruct((L, D_MODEL, B), jnp.float32),
        mesh=mesh,
        compiler_params=pltpu.CompilerParams(needs_layout_passes=False),
        scratch_types=[
            pltpu.VMEM((N_OBS * D_MODEL,), jnp.float32),
            pltpu.VMEM((N_FEAT * D_MODEL,), jnp.float32),
            pltpu.VMEM((D_MODEL,), jnp.float32),
            pltpu.VMEM((D_MODEL,), jnp.float32),
            pltpu.VMEM((2 * BLK,), jnp.int32),
            pltpu.VMEM((2 * BLK,), jnp.int32),
            pltpu.VMEM((2 * BLK,), jnp.float32),
            pltpu.VMEM((BLK * D_MODEL,), jnp.float32),
            pltpu.VMEM((2, D_MODEL, BLK), jnp.float32),
            pltpu.SemaphoreType.DMA,
            pltpu.SemaphoreType.DMA,
            pltpu.SemaphoreType.DMA,
            pltpu.SemaphoreType.DMA,
        ],
    )
    def embed(val_h, obs_h, feat_h, tab_obs_h, tab_feat_h, w_h, bias_h,
              out_h, tab_o, tab_f, w_v, b_v, obs_v, feat_v, val_v,
              skew_v, out_v, sem_in0, sem_in1, sem_out0, sem_out1):
        wid = lax.axis_index("s") * NUM_CORES + lax.axis_index("c")
        lc = wid // NBC
        bq = wid % NBC
        l0 = lc * lc_size
        b0 = bq * bc_size
        sems_in = (sem_in0, sem_in1)
        sems_out = (sem_out0, sem_out1)

        pltpu.sync_copy(tab_obs_h, tab_o)
        pltpu.sync_copy(tab_feat_h, tab_f)
        pltpu.sync_copy(w_h, w_v)
        pltpu.sync_copy(bias_h, b_v)

        w_regs = [w_v[pl.ds(16 * j, 16)] for j in range(NJ)]
        b_regs = [b_v[pl.ds(16 * j, 16)] for j in range(NJ)]

        def bias_body(r, c):
            for j in range(NJ):
                off = r * D_MODEL + 16 * j
                tab_o[pl.ds(off, 16)] = tab_o[pl.ds(off, 16)] + b_regs[j]
            return c

        lax.fori_loop(0, N_OBS, bias_body, 0)

        iota16 = lax.iota(jnp.int32, 16)

        def tok_off(s):
            return (l0 + s // sub) * B + b0 + (s % sub) * BLK

        def in_copies(s, b):
            row0 = tok_off(s)
            return (
                pltpu.make_async_copy(obs_h.at[pl.ds(row0, BLK)],
                                      obs_v.at[pl.ds(b * BLK, BLK)],
                                      sems_in[b]),
                pltpu.make_async_copy(feat_h.at[pl.ds(row0, BLK)],
                                      feat_v.at[pl.ds(b * BLK, BLK)],
                                      sems_in[b]),
                pltpu.make_async_copy(val_h.at[pl.ds(row0, BLK)],
                                      val_v.at[pl.ds(b * BLK, BLK)],
                                      sems_in[b]),
            )

        def out_copy(s, b):
            lg = l0 + s // sub
            bcol = b0 + (s % sub) * BLK
            return pltpu.make_async_copy(
                out_v.at[b],
                out_h.at[lg, :, pl.ds(bcol, BLK)],
                sems_out[b])

        for c in in_copies(0, 0):
            c.start()

        def pair_body(g, carry):
            for b in range(2):
                s = g * 2 + b

                @pl.when(s + 1 < steps)
                def _():
                    for c in in_copies(s + 1, 1 - b):
                        c.start()

                for c in in_copies(s, b):
                    c.wait()

                @pl.when(s >= 2)
                def _():
                    out_copy(s - 2, b).wait()

                def grp_body(gi, carry2):
                    t0b = b * BLK + gi * 16
                    o16 = obs_v[pl.ds(t0b, 16)] * D_MODEL
                    f16 = feat_v[pl.ds(t0b, 16)] * D_MODEL
                    v16 = val_v[pl.ds(t0b, 16)]
                    t0 = gi * 16

                    def load_tok(ob, fb):
                        return (
                            [tab_o[pl.ds(ob + 16 * j, 16)] for j in range(NJ)],
                            [tab_f[pl.ds(fb + 16 * j, 16)] for j in range(NJ)],
                        )

                    PF = 2
                    obase = [None] * 16
                    fbase = [None] * 16
                    for k in range(PF):
                        obase[k] = o16[k]
                        fbase[k] = f16[k]
                    ro, rf = load_tok(obase[0], fbase[0])
                    for k in range(16):
                        if k + PF < 16:
                            obase[k + PF] = o16[k + PF]
                            fbase[k + PF] = f16[k + PF]
                        if k + 1 < 16:
                            ro_n, rf_n = load_tok(obase[k + 1], fbase[k + 1])
                        vb = v16[k]
                        t = t0 + k
                        tb = t * D_MODEL
                        for j in range(NJ):
                            rot = (iota16 + (k + 16 * j)) & (D_MODEL - 1)
                            plsc.store_scatter(
                                skew_v, [rot + tb],
                                ro[j] + rf[j] + vb * w_regs[j])
                        if k + 1 < 16:
                            ro, rf = ro_n, rf_n
                    return carry2

                lax.fori_loop(0, BLK // 16, grp_body, 0)

                NG = 2

                def unskew_body(tg, carry2):
                    tvs = [(tg * 16 * NG + 16 * g + iota16) * D_MODEL
                           for g in range(NG)]
                    rot0 = iota16 & (D_MODEL - 1)
                    cur = [plsc.load_gather(skew_v, [tvs[g] + rot0])
                           for g in range(NG)]
                    for d in range(D_MODEL):
                        if d + 1 < D_MODEL:
                            rot_n = (iota16 + (d + 1)) & (D_MODEL - 1)
                            nxt = [plsc.load_gather(skew_v, [tvs[g] + rot_n])
                                   for g in range(NG)]
                        for g in range(NG):
                            out_v[b, d, pl.ds(tg * 16 * NG + 16 * g, 16)] = (
                                cur[g])
                        if d + 1 < D_MODEL:
                            cur = nxt
                    return carry2

                lax.fori_loop(0, BLK // (16 * NG), unskew_body, 0)

                out_copy(s, b).start()
            return carry

        lax.fori_loop(0, steps // 2, pair_body, 0)
        out_copy(steps - 2, 0).wait()
        out_copy(steps - 1, 1).wait()

    return embed


def kernel(val, obs_idx, feat_idx, W_val, b_val, emb_obs, emb_feat):
    B, L, _ = val.shape
    T = B * L
    val_f = val.reshape(B, L).T.reshape(T).astype(jnp.float32)
    obs_f = obs_idx.T.reshape(T).astype(jnp.int32)
    feat_f = feat_idx.T.reshape(T).astype(jnp.int32)
    w_f = W_val.reshape(D_MODEL).astype(jnp.float32)
    b_f = b_val.reshape(D_MODEL).astype(jnp.float32)
    out_t = _build(B, L)(val_f, obs_f, feat_f,
                         emb_obs.astype(jnp.float32).reshape(N_OBS * D_MODEL),
                         emb_feat.astype(jnp.float32).reshape(N_FEAT * D_MODEL),
                         w_f, b_f)
    return jnp.transpose(out_t, (2, 0, 1))
